# baseline (speedup 1.0000x reference)
import math
import numpy as np
import jax
import jax.numpy as jnp
from jax.experimental import pallas as pl
from jax.experimental.pallas import tpu as pltpu

IN_CH = 5; P = 8; SFH = 64; EMB = 128; HID = 512; HEADS = 4; FFN = 2048
LAYERS = 2; NSPEC = 6; TOPK = 2; NSHARED = 2; NCLS = 2; ALPHA = 1e-4
B = 2; S = 256; T = B * S; DH = HID // HEADS
F2 = FFN // 2
NSTEPS = 2 + 17 * LAYERS
RING = 8
LOOKAHEAD = 7
NBLK = 16 * LAYERS


def _pos_encoding_np():
    pos = np.arange(S, dtype=np.float32)[:, None]
    div = np.exp(np.arange(0, EMB, 2, dtype=np.float32) * (-math.log(10000.0) / EMB))
    pe = np.zeros((S, EMB), np.float32)
    pe[:, 0::2] = np.sin(pos * div)
    pe[:, 1::2] = np.cos(pos * div)
    return np.tile(pe, (B, 1))


MMDT = jnp.float32


def _bdot(a, b):
    return jax.lax.dot_general(
        a.astype(MMDT), b.astype(MMDT),
        (((1,), (0,)), ((), ())), preferred_element_type=jnp.float32)


def _bdot_t(a, b):
    return jax.lax.dot_general(
        a.astype(MMDT), b.astype(MMDT),
        (((1,), (1,)), ((), ())), preferred_element_type=jnp.float32)


def _ln(x, g, b, eps=1e-5):
    m = x.mean(-1, keepdims=True)
    v = ((x - m) ** 2).mean(-1, keepdims=True)
    return (x - m) / jnp.sqrt(v + eps) * g + b


def _gelu(z):
    return 0.5 * z * (1.0 + jax.lax.erf(z * (1.0 / math.sqrt(2.0))))


N_GLOBAL = 12
N_PER_LAYER = 20
LKEYS = ['qkv_w', 'qkv_b', 'out_w', 'out_b', 'n1_g', 'n1_b', 'sp_router',
         'sh_router', 'mn_g', 'mn_b', 'n2_g', 'n2_b',
         'sp_fc1_w', 'sp_fc1_b', 'sp_fc2_w', 'sp_fc2_b',
         'sh_fc1_w', 'sh_fc1_b', 'sh_fc2_w', 'sh_fc2_b']


def _mega_kernel(*refs):
    nin = N_GLOBAL + N_PER_LAYER * LAYERS
    (t_r, sf1, sfb1, sf2, sfb2, pos, pw, pb, rw, rb, cw, cb) = refs[:N_GLOBAL]
    layer_refs = [refs[N_GLOBAL + l * N_PER_LAYER: N_GLOBAL + (l + 1) * N_PER_LAYER]
                  for l in range(LAYERS)]
    recon_o, logits_o, aux_o = refs[nin:nin + 3]
    (h_s, y_s, ybf_s, acc_s, g_s, oh_s, rp_s,
     wr1_s, wr2_s, sem1, sem2) = refs[nin + 3:]

    s_id = pl.program_id(0)

    def _w1_copy(w1a, e, f, slot):
        return pltpu.make_async_copy(
            w1a.at[e, :, pl.ds(f * F2, F2)], wr1_s.at[slot], sem1.at[slot])

    def _w2_copy(w2a, e, f, slot):
        return pltpu.make_async_copy(
            w2a.at[e, pl.ds(f * F2, F2), :], wr2_s.at[slot], sem2.at[slot])

    def _issue_pair(i):
        slot = i % RING
        lblk = i // 16
        rem = i % 16
        for l in range(LAYERS):
            spw1_a = layer_refs[l][12]
            spw2_a = layer_refs[l][14]
            shw1_a = layer_refs[l][16]
            shw2_a = layer_refs[l][18]

            @pl.when((lblk == l) & (rem < 12))
            def _(spw1_a=spw1_a, spw2_a=spw2_a):
                _w1_copy(spw1_a, rem // 2, rem % 2, slot).start()
                _w2_copy(spw2_a, rem // 2, rem % 2, slot).start()

            @pl.when((lblk == l) & (rem >= 12))
            def _(shw1_a=shw1_a, shw2_a=shw2_a):
                j = rem - 12
                _w1_copy(shw1_a, j // 2, j % 2, slot).start()
                _w2_copy(shw2_a, j // 2, j % 2, slot).start()

    @pl.when(s_id == 0)
    def _():
        t = jnp.maximum(t_r[...] @ sf1[...] + sfb1[...], 0.0)
        t = jnp.maximum(t @ sf2[...] + sfb2[...], 0.0)
        t = t + pos[...]
        h_s[...] = t @ pw[...] + pb[...]
        spw1_0 = layer_refs[0][12]
        spw2_0 = layer_refs[0][14]
        for i in range(LOOKAHEAD):
            _w1_copy(spw1_0, i // 2, i % 2, i % RING).start()
            _w2_copy(spw2_0, i // 2, i % 2, i % RING).start()

    def _combine(l):
        mng, mnb, n2g, n2b = layer_refs[l][8:12]
        y = y_s[...]
        m = _ln(y + acc_s[...], mng[...], mnb[...])
        h_s[...] = _ln(y + m, n2g[...], n2b[...])

    for l in range(LAYERS):
        base = 1 + 17 * l
        (qkvw, qkvb, outw, outb, n1g, n1b, spr, shr, mng, mnb, n2g, n2b,
         spw1, spb1, spw2, spb2, shw1, shb1, shw2, shb2) = layer_refs[l]

        @pl.when(s_id == base)
        def _(l=l, qkvw=qkvw, qkvb=qkvb, outw=outw, outb=outb, n1g=n1g,
              n1b=n1b, spr=spr, shr=shr):
            if l > 0:
                _combine(l - 1)
            oh_acc = None
            rp_acc = None
            scale = 1.0 / math.sqrt(DH)
            for b in range(B):
                x = h_s[b * S:(b + 1) * S, :]
                qkv = _bdot(x, qkvw[...]) + qkvb[...]
                outs = []
                for hd in range(HEADS):
                    q = qkv[:, hd * DH:(hd + 1) * DH]
                    k = qkv[:, HID + hd * DH: HID + (hd + 1) * DH]
                    v = qkv[:, 2 * HID + hd * DH: 2 * HID + (hd + 1) * DH]
                    sc = _bdot_t(q, k) * scale
                    a = jax.nn.softmax(sc, axis=-1)
                    outs.append(_bdot(a, v))
                o = jnp.concatenate(outs, axis=1)
                att = _bdot(o, outw[...]) + outb[...]
                y = _ln(x + att, n1g[...], n1b[...])
                y_s[b * S:(b + 1) * S, :] = y
                ybf_s[b * S:(b + 1) * S, :] = y.astype(MMDT)

                pr = jax.nn.softmax(y @ spr[...], axis=-1)
                p1 = jnp.full((S, 1), -1.0, jnp.float32)
                i1 = jnp.zeros((S, 1), jnp.int32)
                for e in range(NSPEC):
                    pe = pr[:, e:e + 1]
                    upd = pe > p1
                    i1 = jnp.where(upd, e, i1)
                    p1 = jnp.where(upd, pe, p1)
                p2 = jnp.full((S, 1), -1.0, jnp.float32)
                i2 = jnp.zeros((S, 1), jnp.int32)
                for e in range(NSPEC):
                    pe = pr[:, e:e + 1]
                    upd = (pe > p2) & (i1 != e)
                    i2 = jnp.where(upd, e, i2)
                    p2 = jnp.where(upd, pe, p2)
                den = p1 + p2 + 1e-9
                w1n = p1 / den
                w2n = p2 / den
                for e in range(NSPEC):
                    ge = jnp.where(i1 == e, w1n, 0.0) + \
                        jnp.where(i2 == e, w2n, 0.0)
                    g_s[e, b * S:(b + 1) * S, :] = ge
                sh_p = jax.nn.softmax(y @ shr[...], axis=-1)
                for e in range(NSHARED):
                    g_s[NSPEC + e, b * S:(b + 1) * S, :] = sh_p[:, e:e + 1]

                ohb = jnp.concatenate(
                    [jnp.sum(((i1 == e) | (i2 == e)).astype(jnp.float32),
                             axis=0, keepdims=True) for e in range(NSPEC)]
                    + [jnp.zeros((1, 2), jnp.float32)], axis=1)
                rpb = jnp.concatenate(
                    [jnp.sum(pr, axis=0, keepdims=True),
                     jnp.zeros((1, 2), jnp.float32)], axis=1)
                oh_acc = ohb if oh_acc is None else oh_acc + ohb
                rp_acc = rpb if rp_acc is None else rp_acc + rpb
            oh_s[l:l + 1, :] = oh_acc
            rp_s[l:l + 1, :] = rp_acc
            acc_s[...] = jnp.zeros((T, HID), jnp.float32)

        def _expert_step(w1a, b1r, w2a, b2r, start, gate_base, blk_base):
            rel = s_id - start
            f0 = (rel % 2) == 0
            e = rel // 2
            f = rel % 2
            i_now = blk_base + rel
            slot = i_now % RING
            _w1_copy(w1a, e, f, slot).wait()
            _w2_copy(w2a, e, f, slot).wait()
            w1 = wr1_s[pl.ds(slot, 1)][0]
            w2 = wr2_s[pl.ds(slot, 1)][0]
            z = _bdot(ybf_s[...], w1) + b1r[0]
            h1 = _gelu(z)
            part = _bdot(h1, w2)
            g = g_s[pl.ds(gate_base + e, 1), :, :][0]
            bias = jnp.where(f0, 1.0, 0.0) * b2r[0]
            acc_s[...] += (part + bias) * g
            i_next = i_now + LOOKAHEAD

            @pl.when(i_next < NBLK)
            def _():
                _issue_pair(i_next)

        @pl.when((s_id >= base + 1) & (s_id <= base + 12))
        def _(spw1=spw1, spb1=spb1, spw2=spw2, spb2=spb2, base=base, l=l):
            _expert_step(spw1, spb1, spw2, spb2, base + 1, 0, 16 * l)

        @pl.when((s_id >= base + 13) & (s_id <= base + 16))
        def _(shw1=shw1, shb1=shb1, shw2=shw2, shb2=shb2, base=base, l=l):
            _expert_step(shw1, shb1, shw2, shb2, base + 13, NSPEC, 16 * l + 12)

    @pl.when(s_id == NSTEPS - 1)
    def _():
        _combine(LAYERS - 1)
        h = h_s[...]
        recon_o[...] = h @ rw[...] + rb[...]
        pooled = jnp.concatenate(
            [jnp.mean(h[b * S:(b + 1) * S, :], axis=0, keepdims=True)
             for b in range(B)], axis=0)
        logits_o[...] = pooled @ cw[...] + cb[...]
        ohm = oh_s[...] / float(T)
        rpm = rp_s[...] / float(T)
        aux_o[...] = jnp.sum(ohm * rpm).reshape(1, 1)


def _c2(s):
    return (0, 0)


def _c3(s):
    return (0, 0, 0)


def _sp_maps(l):
    st = 2 + 17 * l

    def w1(s):
        j = jnp.clip(s - st, 0, 11)
        return (j // 2, 0, j % 2)

    def b1(s):
        j = jnp.clip(s - st, 0, 11)
        return (j // 2, 0, j % 2)

    def w2(s):
        j = jnp.clip(s - st, 0, 11)
        return (j // 2, j % 2, 0)

    def b2(s):
        j = jnp.clip(s - st, 0, 11)
        return (j // 2, 0, 0)
    return w1, b1, w2, b2


def _sh_maps(l):
    st = 14 + 17 * l

    def w1(s):
        j = jnp.clip(s - st, 0, 3)
        return (j // 2, 0, j % 2)

    def b1(s):
        j = jnp.clip(s - st, 0, 3)
        return (j // 2, 0, j % 2)

    def w2(s):
        j = jnp.clip(s - st, 0, 3)
        return (j // 2, j % 2, 0)

    def b2(s):
        j = jnp.clip(s - st, 0, 3)
        return (j // 2, 0, 0)
    return w1, b1, w2, b2


def kernel(x, params):
    nH, nW = 128 // P, 128 // P
    t = x.reshape(B, IN_CH, nH, P, nW, P).transpose(0, 1, 2, 4, 3, 5)
    t = t.reshape(B, IN_CH, nH * nW, P, P).transpose(0, 2, 1, 3, 4)
    t = t.reshape(T, IN_CH * P * P)
    pos = jnp.asarray(_pos_encoding_np())
    p = params

    operands = [t, p['sf_w1'], p['sf_b1'].reshape(1, -1), p['sf_w2'],
                p['sf_b2'].reshape(1, -1), pos, p['proj_w'],
                p['proj_b'].reshape(1, -1), p['recon_w'],
                p['recon_b'].reshape(1, -1), p['cls_w'],
                p['cls_b'].reshape(1, -1)]
    in_specs = [
        pl.BlockSpec((T, IN_CH * P * P), _c2),
        pl.BlockSpec((IN_CH * P * P, SFH), _c2),
        pl.BlockSpec((1, SFH), _c2),
        pl.BlockSpec((SFH, EMB), _c2),
        pl.BlockSpec((1, EMB), _c2),
        pl.BlockSpec((T, EMB), _c2),
        pl.BlockSpec((EMB, HID), _c2),
        pl.BlockSpec((1, HID), _c2),
        pl.BlockSpec((HID, EMB), _c2),
        pl.BlockSpec((1, EMB), _c2),
        pl.BlockSpec((HID, NCLS), _c2),
        pl.BlockSpec((1, NCLS), _c2),
    ]
    for l, L in enumerate(p['layers']):
        spm = _sp_maps(l)
        shm = _sh_maps(l)
        operands += [
            L['qkv_w'], L['qkv_b'].reshape(1, -1), L['out_w'],
            L['out_b'].reshape(1, -1), L['n1_g'].reshape(1, -1),
            L['n1_b'].reshape(1, -1), L['sp_router'], L['sh_router'],
            L['mn_g'].reshape(1, -1), L['mn_b'].reshape(1, -1),
            L['n2_g'].reshape(1, -1), L['n2_b'].reshape(1, -1),
            L['sp_fc1_w'], L['sp_fc1_b'].reshape(NSPEC, 1, FFN),
            L['sp_fc2_w'], L['sp_fc2_b'].reshape(NSPEC, 1, HID),
            L['sh_fc1_w'], L['sh_fc1_b'].reshape(NSHARED, 1, FFN),
            L['sh_fc2_w'], L['sh_fc2_b'].reshape(NSHARED, 1, HID),
        ]
        in_specs += [
            pl.BlockSpec((HID, 3 * HID), _c2),
            pl.BlockSpec((1, 3 * HID), _c2),
            pl.BlockSpec((HID, HID), _c2),
            pl.BlockSpec((1, HID), _c2),
            pl.BlockSpec((1, HID), _c2),
            pl.BlockSpec((1, HID), _c2),
            pl.BlockSpec((HID, NSPEC), _c2),
            pl.BlockSpec((HID, NSHARED), _c2),
            pl.BlockSpec((1, HID), _c2),
            pl.BlockSpec((1, HID), _c2),
            pl.BlockSpec((1, HID), _c2),
            pl.BlockSpec((1, HID), _c2),
            pl.BlockSpec(memory_space=pl.ANY),
            pl.BlockSpec((1, 1, F2), spm[1]),
            pl.BlockSpec(memory_space=pl.ANY),
            pl.BlockSpec((1, 1, HID), spm[3]),
            pl.BlockSpec(memory_space=pl.ANY),
            pl.BlockSpec((1, 1, F2), shm[1]),
            pl.BlockSpec(memory_space=pl.ANY),
            pl.BlockSpec((1, 1, HID), shm[3]),
        ]

    recon, logits, auxm = pl.pallas_call(
        _mega_kernel,
        grid=(NSTEPS,),
        in_specs=in_specs,
        out_specs=[
            pl.BlockSpec((T, EMB), _c2),
            pl.BlockSpec((B, NCLS), _c2),
            pl.BlockSpec((1, 1), _c2),
        ],
        out_shape=[
            jax.ShapeDtypeStruct((T, EMB), jnp.float32),
            jax.ShapeDtypeStruct((B, NCLS), jnp.float32),
            jax.ShapeDtypeStruct((1, 1), jnp.float32),
        ],
        scratch_shapes=[
            pltpu.VMEM((T, HID), jnp.float32),
            pltpu.VMEM((T, HID), jnp.float32),
            pltpu.VMEM((T, HID), MMDT),
            pltpu.VMEM((T, HID), jnp.float32),
            pltpu.VMEM((NSPEC + NSHARED, T, 1), jnp.float32),
            pltpu.VMEM((LAYERS, 8), jnp.float32),
            pltpu.VMEM((LAYERS, 8), jnp.float32),
            pltpu.VMEM((RING, HID, F2), jnp.float32),
            pltpu.VMEM((RING, F2, HID), jnp.float32),
            pltpu.SemaphoreType.DMA((RING,)),
            pltpu.SemaphoreType.DMA((RING,)),
        ],
    )(*operands)

    aux = ALPHA * NSPEC * auxm.reshape(())
    return logits, recon.reshape(B, S, EMB), aux


# speedup vs baseline: 1.0090x; 1.0090x over previous
import math
import numpy as np
import jax
import jax.numpy as jnp
from jax.experimental import pallas as pl
from jax.experimental.pallas import tpu as pltpu

IN_CH = 5; P = 8; SFH = 64; EMB = 128; HID = 512; HEADS = 4; FFN = 2048
LAYERS = 2; NSPEC = 6; TOPK = 2; NSHARED = 2; NCLS = 2; ALPHA = 1e-4
B = 2; S = 256; T = B * S; DH = HID // HEADS
F2 = FFN // 2
NSTEPS = 2 + 17 * LAYERS
RING = 8
LOOKAHEAD = 7
NBLK = 16 * LAYERS


def _pos_encoding_np():
    pos = np.arange(S, dtype=np.float32)[:, None]
    div = np.exp(np.arange(0, EMB, 2, dtype=np.float32) * (-math.log(10000.0) / EMB))
    pe = np.zeros((S, EMB), np.float32)
    pe[:, 0::2] = np.sin(pos * div)
    pe[:, 1::2] = np.cos(pos * div)
    return np.tile(pe, (B, 1))


def _bdot(a, b):
    return jax.lax.dot_general(
        a, b, (((1,), (0,)), ((), ())), preferred_element_type=jnp.float32)


def _bdot_t(a, b):
    return jax.lax.dot_general(
        a, b, (((1,), (1,)), ((), ())), preferred_element_type=jnp.float32)


def _ln(x, g, b, eps=1e-5):
    m = x.mean(-1, keepdims=True)
    v = ((x - m) ** 2).mean(-1, keepdims=True)
    return (x - m) / jnp.sqrt(v + eps) * g + b


def _gelu(z):
    return 0.5 * z * (1.0 + jax.lax.erf(z * (1.0 / math.sqrt(2.0))))


N_GLOBAL = 12
N_PER_LAYER = 20


def _mega_kernel(*refs):
    nin = N_GLOBAL + N_PER_LAYER * LAYERS
    (t_r, sf1, sfb1, sf2, sfb2, pos, pw, pb, rw, rb, cw, cb) = refs[:N_GLOBAL]
    layer_refs = [refs[N_GLOBAL + l * N_PER_LAYER: N_GLOBAL + (l + 1) * N_PER_LAYER]
                  for l in range(LAYERS)]
    recon_o, logits_o, aux_o = refs[nin:nin + 3]
    (h_s, y_s, acc_s, g_s, oh_s, rp_s,
     wr1_s, wr2_s, sem1, sem2) = refs[nin + 3:]

    s_id = pl.program_id(0)

    def _w1_copy(w1a, e, f, slot):
        return pltpu.make_async_copy(
            w1a.at[e, :, pl.ds(f * F2, F2)], wr1_s.at[slot], sem1.at[slot])

    def _w2_copy(w2a, e, f, slot):
        return pltpu.make_async_copy(
            w2a.at[e, pl.ds(f * F2, F2), :], wr2_s.at[slot], sem2.at[slot])

    def _issue_pair(i):
        slot = i % RING
        lblk = i // 16
        rem = i % 16
        for l in range(LAYERS):
            spw1_a = layer_refs[l][12]
            spw2_a = layer_refs[l][14]
            shw1_a = layer_refs[l][16]
            shw2_a = layer_refs[l][18]

            @pl.when((lblk == l) & (rem < 12))
            def _(spw1_a=spw1_a, spw2_a=spw2_a):
                _w1_copy(spw1_a, rem // 2, rem % 2, slot).start()
                _w2_copy(spw2_a, rem // 2, rem % 2, slot).start()

            @pl.when((lblk == l) & (rem >= 12))
            def _(shw1_a=shw1_a, shw2_a=shw2_a):
                j = rem - 12
                _w1_copy(shw1_a, j // 2, j % 2, slot).start()
                _w2_copy(shw2_a, j // 2, j % 2, slot).start()

    @pl.when(s_id == 0)
    def _():
        t = jnp.maximum(t_r[...] @ sf1[...] + sfb1[...], 0.0)
        t = jnp.maximum(t @ sf2[...] + sfb2[...], 0.0)
        t = t + pos[...]
        h_s[...] = t @ pw[...] + pb[...]
        spw1_0 = layer_refs[0][12]
        spw2_0 = layer_refs[0][14]
        for i in range(LOOKAHEAD):
            _w1_copy(spw1_0, i // 2, i % 2, i % RING).start()
            _w2_copy(spw2_0, i // 2, i % 2, i % RING).start()

    def _combine(l):
        mng, mnb, n2g, n2b = layer_refs[l][8:12]
        y = y_s[...]
        m = _ln(y + acc_s[...], mng[...], mnb[...])
        h_s[...] = _ln(y + m, n2g[...], n2b[...])

    for l in range(LAYERS):
        base = 1 + 17 * l
        (qkvw, qkvb, outw, outb, n1g, n1b, spr, shr, mng, mnb, n2g, n2b,
         spw1, spb1, spw2, spb2, shw1, shb1, shw2, shb2) = layer_refs[l]

        @pl.when(s_id == base)
        def _(l=l, qkvw=qkvw, qkvb=qkvb, outw=outw, outb=outb, n1g=n1g,
              n1b=n1b, spr=spr, shr=shr):
            if l > 0:
                _combine(l - 1)
            oh_acc = None
            rp_acc = None
            scale = 1.0 / math.sqrt(DH)
            for b in range(B):
                x = h_s[b * S:(b + 1) * S, :]
                qkv = _bdot(x, qkvw[...]) + qkvb[...]
                outs = []
                for hd in range(HEADS):
                    q = qkv[:, hd * DH:(hd + 1) * DH]
                    k = qkv[:, HID + hd * DH: HID + (hd + 1) * DH]
                    v = qkv[:, 2 * HID + hd * DH: 2 * HID + (hd + 1) * DH]
                    sc = _bdot_t(q, k) * scale
                    a = jax.nn.softmax(sc, axis=-1)
                    outs.append(_bdot(a, v))
                o = jnp.concatenate(outs, axis=1)
                att = _bdot(o, outw[...]) + outb[...]
                y = _ln(x + att, n1g[...], n1b[...])
                y_s[b * S:(b + 1) * S, :] = y

                pr = jax.nn.softmax(y @ spr[...], axis=-1)
                p1 = jnp.full((S, 1), -1.0, jnp.float32)
                i1 = jnp.zeros((S, 1), jnp.int32)
                for e in range(NSPEC):
                    pe = pr[:, e:e + 1]
                    upd = pe > p1
                    i1 = jnp.where(upd, e, i1)
                    p1 = jnp.where(upd, pe, p1)
                p2 = jnp.full((S, 1), -1.0, jnp.float32)
                i2 = jnp.zeros((S, 1), jnp.int32)
                for e in range(NSPEC):
                    pe = pr[:, e:e + 1]
                    upd = (pe > p2) & (i1 != e)
                    i2 = jnp.where(upd, e, i2)
                    p2 = jnp.where(upd, pe, p2)
                den = p1 + p2 + 1e-9
                w1n = p1 / den
                w2n = p2 / den
                for e in range(NSPEC):
                    ge = jnp.where(i1 == e, w1n, 0.0) + \
                        jnp.where(i2 == e, w2n, 0.0)
                    g_s[e, b * S:(b + 1) * S, :] = ge
                sh_p = jax.nn.softmax(y @ shr[...], axis=-1)
                for e in range(NSHARED):
                    g_s[NSPEC + e, b * S:(b + 1) * S, :] = sh_p[:, e:e + 1]

                ohb = jnp.concatenate(
                    [jnp.sum(((i1 == e) | (i2 == e)).astype(jnp.float32),
                             axis=0, keepdims=True) for e in range(NSPEC)]
                    + [jnp.zeros((1, 2), jnp.float32)], axis=1)
                rpb = jnp.concatenate(
                    [jnp.sum(pr, axis=0, keepdims=True),
                     jnp.zeros((1, 2), jnp.float32)], axis=1)
                oh_acc = ohb if oh_acc is None else oh_acc + ohb
                rp_acc = rpb if rp_acc is None else rp_acc + rpb
            oh_s[l:l + 1, :] = oh_acc
            rp_s[l:l + 1, :] = rp_acc
            acc_s[...] = jnp.zeros((T, HID), jnp.float32)

        def _expert_step(w1a, b1r, w2a, b2r, start, gate_base, blk_base):
            rel = s_id - start
            f0 = (rel % 2) == 0
            e = rel // 2
            f = rel % 2
            i_now = blk_base + rel
            slot = i_now % RING
            _w1_copy(w1a, e, f, slot).wait()
            _w2_copy(w2a, e, f, slot).wait()
            w1 = wr1_s[pl.ds(slot, 1)][0]
            w2 = wr2_s[pl.ds(slot, 1)][0]
            z = _bdot(y_s[...], w1) + b1r[0]
            h1 = _gelu(z)
            part = _bdot(h1, w2)
            g = g_s[pl.ds(gate_base + e, 1), :, :][0]
            bias = jnp.where(f0, 1.0, 0.0) * b2r[0]
            acc_s[...] += (part + bias) * g
            i_next = i_now + LOOKAHEAD

            @pl.when(i_next < NBLK)
            def _():
                _issue_pair(i_next)

        @pl.when((s_id >= base + 1) & (s_id <= base + 12))
        def _(spw1=spw1, spb1=spb1, spw2=spw2, spb2=spb2, base=base, l=l):
            _expert_step(spw1, spb1, spw2, spb2, base + 1, 0, 16 * l)

        @pl.when((s_id >= base + 13) & (s_id <= base + 16))
        def _(shw1=shw1, shb1=shb1, shw2=shw2, shb2=shb2, base=base, l=l):
            _expert_step(shw1, shb1, shw2, shb2, base + 13, NSPEC, 16 * l + 12)

    @pl.when(s_id == NSTEPS - 1)
    def _():
        _combine(LAYERS - 1)
        h = h_s[...]
        recon_o[...] = h @ rw[...] + rb[...]
        pooled = jnp.concatenate(
            [jnp.mean(h[b * S:(b + 1) * S, :], axis=0, keepdims=True)
             for b in range(B)], axis=0)
        logits_o[...] = pooled @ cw[...] + cb[...]
        ohm = oh_s[...] / float(T)
        rpm = rp_s[...] / float(T)
        aux_o[...] = jnp.sum(ohm * rpm).reshape(1, 1)


def _c2(s):
    return (0, 0)


def _sp_maps(l):
    st = 2 + 17 * l

    def w1(s):
        j = jnp.clip(s - st, 0, 11)
        return (j // 2, 0, j % 2)

    def b1(s):
        j = jnp.clip(s - st, 0, 11)
        return (j // 2, 0, j % 2)

    def w2(s):
        j = jnp.clip(s - st, 0, 11)
        return (j // 2, j % 2, 0)

    def b2(s):
        j = jnp.clip(s - st, 0, 11)
        return (j // 2, 0, 0)
    return w1, b1, w2, b2


def _sh_maps(l):
    st = 14 + 17 * l

    def w1(s):
        j = jnp.clip(s - st, 0, 3)
        return (j // 2, 0, j % 2)

    def b1(s):
        j = jnp.clip(s - st, 0, 3)
        return (j // 2, 0, j % 2)

    def w2(s):
        j = jnp.clip(s - st, 0, 3)
        return (j // 2, j % 2, 0)

    def b2(s):
        j = jnp.clip(s - st, 0, 3)
        return (j // 2, 0, 0)
    return w1, b1, w2, b2


def kernel(x, params):
    nH, nW = 128 // P, 128 // P
    t = x.reshape(B, IN_CH, nH, P, nW, P).transpose(0, 1, 2, 4, 3, 5)
    t = t.reshape(B, IN_CH, nH * nW, P, P).transpose(0, 2, 1, 3, 4)
    t = t.reshape(T, IN_CH * P * P)
    pos = jnp.asarray(_pos_encoding_np())
    p = params

    operands = [t, p['sf_w1'], p['sf_b1'].reshape(1, -1), p['sf_w2'],
                p['sf_b2'].reshape(1, -1), pos, p['proj_w'],
                p['proj_b'].reshape(1, -1), p['recon_w'],
                p['recon_b'].reshape(1, -1), p['cls_w'],
                p['cls_b'].reshape(1, -1)]
    in_specs = [
        pl.BlockSpec((T, IN_CH * P * P), _c2),
        pl.BlockSpec((IN_CH * P * P, SFH), _c2),
        pl.BlockSpec((1, SFH), _c2),
        pl.BlockSpec((SFH, EMB), _c2),
        pl.BlockSpec((1, EMB), _c2),
        pl.BlockSpec((T, EMB), _c2),
        pl.BlockSpec((EMB, HID), _c2),
        pl.BlockSpec((1, HID), _c2),
        pl.BlockSpec((HID, EMB), _c2),
        pl.BlockSpec((1, EMB), _c2),
        pl.BlockSpec((HID, NCLS), _c2),
        pl.BlockSpec((1, NCLS), _c2),
    ]
    for l, L in enumerate(p['layers']):
        spm = _sp_maps(l)
        shm = _sh_maps(l)
        operands += [
            L['qkv_w'], L['qkv_b'].reshape(1, -1), L['out_w'],
            L['out_b'].reshape(1, -1), L['n1_g'].reshape(1, -1),
            L['n1_b'].reshape(1, -1), L['sp_router'], L['sh_router'],
            L['mn_g'].reshape(1, -1), L['mn_b'].reshape(1, -1),
            L['n2_g'].reshape(1, -1), L['n2_b'].reshape(1, -1),
            L['sp_fc1_w'], L['sp_fc1_b'].reshape(NSPEC, 1, FFN),
            L['sp_fc2_w'], L['sp_fc2_b'].reshape(NSPEC, 1, HID),
            L['sh_fc1_w'], L['sh_fc1_b'].reshape(NSHARED, 1, FFN),
            L['sh_fc2_w'], L['sh_fc2_b'].reshape(NSHARED, 1, HID),
        ]
        in_specs += [
            pl.BlockSpec((HID, 3 * HID), _c2),
            pl.BlockSpec((1, 3 * HID), _c2),
            pl.BlockSpec((HID, HID), _c2),
            pl.BlockSpec((1, HID), _c2),
            pl.BlockSpec((1, HID), _c2),
            pl.BlockSpec((1, HID), _c2),
            pl.BlockSpec((HID, NSPEC), _c2),
            pl.BlockSpec((HID, NSHARED), _c2),
            pl.BlockSpec((1, HID), _c2),
            pl.BlockSpec((1, HID), _c2),
            pl.BlockSpec((1, HID), _c2),
            pl.BlockSpec((1, HID), _c2),
            pl.BlockSpec(memory_space=pl.ANY),
            pl.BlockSpec((1, 1, F2), spm[1]),
            pl.BlockSpec(memory_space=pl.ANY),
            pl.BlockSpec((1, 1, HID), spm[3]),
            pl.BlockSpec(memory_space=pl.ANY),
            pl.BlockSpec((1, 1, F2), shm[1]),
            pl.BlockSpec(memory_space=pl.ANY),
            pl.BlockSpec((1, 1, HID), shm[3]),
        ]

    recon, logits, auxm = pl.pallas_call(
        _mega_kernel,
        grid=(NSTEPS,),
        in_specs=in_specs,
        out_specs=[
            pl.BlockSpec((T, EMB), _c2),
            pl.BlockSpec((B, NCLS), _c2),
            pl.BlockSpec((1, 1), _c2),
        ],
        out_shape=[
            jax.ShapeDtypeStruct((T, EMB), jnp.float32),
            jax.ShapeDtypeStruct((B, NCLS), jnp.float32),
            jax.ShapeDtypeStruct((1, 1), jnp.float32),
        ],
        scratch_shapes=[
            pltpu.VMEM((T, HID), jnp.float32),
            pltpu.VMEM((T, HID), jnp.float32),
            pltpu.VMEM((T, HID), jnp.float32),
            pltpu.VMEM((NSPEC + NSHARED, T, 1), jnp.float32),
            pltpu.VMEM((LAYERS, 8), jnp.float32),
            pltpu.VMEM((LAYERS, 8), jnp.float32),
            pltpu.VMEM((RING, HID, F2), jnp.float32),
            pltpu.VMEM((RING, F2, HID), jnp.float32),
            pltpu.SemaphoreType.DMA((RING,)),
            pltpu.SemaphoreType.DMA((RING,)),
        ],
    )(*operands)

    aux = ALPHA * NSPEC * auxm.reshape(())
    return logits, recon.reshape(B, S, EMB), aux
